# Initial kernel scaffold; baseline (speedup 1.0000x reference)
import jax
import jax.numpy as jnp
from jax.experimental import pallas as pl


def kernel(pillars, pillar_indices, W, gamma, beta):
    raise NotImplementedError("write your pallas kernel here")


# speedup vs baseline: 1.5356x; 1.5356x over previous
import functools

import jax
import jax.numpy as jnp
from jax.experimental import pallas as pl

B, P, N, F, C, Xn, Yn = 8, 12000, 32, 9, 64, 144, 496
BP = B * P
NF = N * F
J = Xn * Yn
JW = Xn * Xn
R = 1200
G = BP // R
JT = 768
NJT = J // JT
NJW = JW // JT


def _encode_body(v_ref, m_ref, feat_ref, s1_ref, s2_ref):
    i = pl.program_id(0)
    v = v_ref[...]
    xf = jnp.dot(v, m_ref[...], preferred_element_type=jnp.float32)
    t = xf
    s = 1024
    while s >= 64:
        t = jnp.maximum(t[:, :s], t[:, s:2 * s])
        s //= 2
    feat_ref[...] = t
    ones = jnp.ones((8, R), jnp.float32)
    s1 = jnp.dot(ones, v, preferred_element_type=jnp.float32)
    s2 = jax.lax.dot_general(v, v, (((0,), (0,)), ((), ())),
                             preferred_element_type=jnp.float32)

    @pl.when(i == 0)
    def _():
        s1_ref[...] = jnp.zeros_like(s1_ref)
        s2_ref[...] = jnp.zeros_like(s2_ref)

    s1_ref[...] += s1
    s2_ref[...] += s2


def _canvas_body(g_ref, valid_ref, a_ref, b_ref, out_ref):
    jt = pl.program_id(1)

    @pl.when(jt < NJW)
    def _():
        g = g_ref[...]
        gt = g.T
        y = jnp.maximum(gt * a_ref[...] + b_ref[...], 0.0)
        out_ref[0] = y * valid_ref[0]

    @pl.when(jt >= NJW)
    def _():
        out_ref[...] = jnp.zeros_like(out_ref)


def kernel(pillars, pillar_indices, W, gamma, beta):
    f32 = jnp.float32
    x2d = pillars.reshape(BP, NF)
    M = jnp.kron(jnp.eye(N, dtype=f32), W.T)

    feat, s1, s2 = pl.pallas_call(
        _encode_body,
        grid=(G,),
        in_specs=[pl.BlockSpec((R, NF), lambda i: (i, 0)),
                  pl.BlockSpec((NF, N * C), lambda i: (0, 0))],
        out_specs=[pl.BlockSpec((R, C), lambda i: (i, 0)),
                   pl.BlockSpec((8, NF), lambda i: (0, 0)),
                   pl.BlockSpec((NF, NF), lambda i: (0, 0))],
        out_shape=[jax.ShapeDtypeStruct((BP, C), f32),
                   jax.ShapeDtypeStruct((8, NF), f32),
                   jax.ShapeDtypeStruct((NF, NF), f32)],
    )(x2d, M)

    Mtot = float(B * P * N)
    s1f = s1[0].reshape(N, F).sum(0)
    s2f = jnp.einsum('aiaj->ij', s2.reshape(N, F, N, F))
    mean = (W @ s1f) / Mtot
    ex2 = jnp.einsum('cf,fg,cg->c', W, s2f, W) / Mtot
    var = ex2 - mean * mean
    a = gamma / jnp.sqrt(var + 1e-5)
    bb = beta - mean * a

    col = jnp.clip(pillar_indices[:, :, 2], 0, Xn - 1)
    lin = pillar_indices[:, :, 1] * Xn + col

    p_iota = jnp.broadcast_to(jnp.arange(P, dtype=jnp.int32), (B, P))
    b_iota = jnp.arange(B, dtype=jnp.int32)[:, None]
    pj = jnp.full((B, JW), -1, jnp.int32).at[b_iota, lin].max(p_iota)
    validm = pj >= 0
    jg = jnp.arange(JW, dtype=jnp.int32) % P
    gidx = jnp.where(validm, pj, jg[None, :])
    featb = feat.reshape(B, P, C)
    gathered = jnp.take_along_axis(featb, gidx[:, :, None], axis=1)
    gathered2d = gathered.reshape(B * JW, C)
    validf = validm.astype(f32).reshape(B * NJW, 1, JT)

    canvas = pl.pallas_call(
        _canvas_body,
        grid=(B, NJT),
        in_specs=[pl.BlockSpec((JT, C), lambda b, j: (b * NJW + jnp.minimum(j, NJW - 1), 0)),
                  pl.BlockSpec((1, 1, JT), lambda b, j: (b * NJW + jnp.minimum(j, NJW - 1), 0, 0)),
                  pl.BlockSpec((C, 1), lambda b, j: (0, 0)),
                  pl.BlockSpec((C, 1), lambda b, j: (0, 0))],
        out_specs=pl.BlockSpec((1, C, JT), lambda b, j: (b, 0, j)),
        out_shape=jax.ShapeDtypeStruct((B, C, J), f32),
    )(gathered2d, validf, a.reshape(C, 1), bb.reshape(C, 1))

    return canvas.reshape(B, C, Xn, Yn)


# speedup vs baseline: 1.8528x; 1.2065x over previous
import functools

import jax
import jax.numpy as jnp
from jax import lax
from jax.experimental import pallas as pl
from jax.experimental.pallas import tpu as pltpu
from jax.experimental.pallas import tpu_sc as plsc

B, P, N, F, C, Xn, Yn = 8, 12000, 32, 9, 64, 144, 496
BP = B * P
NF = N * F
J = Xn * Yn
JW = Xn * Xn
R = 1200
G = BP // R
JT = 768
NJT = J // JT
NJW = JW // JT
JG = JW // 8
GW = 648
GC = 72


def _encode_body(v_ref, m_ref, feat_ref, s1_ref, s2_ref):
    i = pl.program_id(0)
    v = v_ref[...]
    xf = jnp.dot(v, m_ref[...], preferred_element_type=jnp.float32)
    t = xf
    s = 1024
    while s >= 64:
        t = jnp.maximum(t[:, :s], t[:, s:2 * s])
        s //= 2
    feat_ref[...] = t
    ones = jnp.ones((8, R), jnp.float32)
    s1 = jnp.dot(ones, v, preferred_element_type=jnp.float32)
    s2 = jax.lax.dot_general(v, v, (((0,), (0,)), ((), ())),
                             preferred_element_type=jnp.float32)

    @pl.when(i == 0)
    def _():
        s1_ref[...] = jnp.zeros_like(s1_ref)
        s2_ref[...] = jnp.zeros_like(s2_ref)

    s1_ref[...] += s1
    s2_ref[...] += s2


def _sc_body(lin_hbm, feat_hbm, gath_hbm, valid_hbm,
             lin_v, table_v, idx_v, valid_v, gbuf_v, sem):
    wid = lax.axis_index("s") * 2 + lax.axis_index("c")
    lanes = lax.iota(jnp.int32, 16)
    for half in range(2):
        t = wid + half * 32
        b = t // 8
        grp = t - b * 8
        lo = grp * JG
        pltpu.sync_copy(lin_hbm.at[pl.ds(b * P, P)], lin_v)

        def _init(i, carry):
            table_v[pl.ds(i * 16, 16)] = jnp.full((16,), -1.0, jnp.float32)
            return carry
        lax.fori_loop(0, (16 * JG) // 16, _init, 0)

        def _scat(v, carry):
            linv = lin_v[pl.ds(v * 16, 16)]
            off = linv - lo
            mask = (off >= 0) & (off < JG)
            idx = lanes * JG + jnp.clip(off, 0, JG - 1)
            pval = (v * 16 + lanes).astype(jnp.float32)
            plsc.store_scatter(table_v, [idx], pval, mask=mask)
            return carry
        lax.fori_loop(0, P // 16, _scat, 0)

        def _merge(i, carry):
            joff = i * 16
            acc = table_v[pl.ds(joff, 16)]
            for l in range(1, 16):
                acc = jnp.maximum(acc, table_v[pl.ds(l * JG + joff, 16)])
            valid_v[pl.ds(joff, 16)] = jnp.where(acc >= 0.0, 1.0, 0.0)
            spread = lax.rem(lo + joff + lanes, P)
            gidx = b * P + jnp.where(acc >= 0.0, acc.astype(jnp.int32), spread)
            idx_v[pl.ds(joff, 16)] = gidx
            return carry
        lax.fori_loop(0, JG // 16, _merge, 0)

        for w in range(JG // GW):
            base = w * GW
            cps = []
            for k in range(GW // GC):
                cp = pltpu.make_async_copy(
                    feat_hbm.at[idx_v.at[pl.ds(base + k * GC, GC)]],
                    gbuf_v.at[pl.ds(k * GC, GC)], sem)
                cp.start()
                cps.append(cp)
            for cp in cps:
                cp.wait()
            pltpu.sync_copy(gbuf_v, gath_hbm.at[pl.ds(b * JW + lo + base, GW)])

        pltpu.sync_copy(valid_v, valid_hbm.at[pl.ds(b * JW + lo, JG)])


def _canvas_body(g_ref, valid_ref, a_ref, b_ref, out_ref):
    jt = pl.program_id(1)

    @pl.when(jt < NJW)
    def _():
        g = g_ref[...]
        gt = g.T
        y = jnp.maximum(gt * a_ref[...] + b_ref[...], 0.0)
        out_ref[0] = y * valid_ref[0]

    @pl.when(jt >= NJW)
    def _():
        out_ref[...] = jnp.zeros_like(out_ref)


def kernel(pillars, pillar_indices, W, gamma, beta):
    f32 = jnp.float32
    x2d = pillars.reshape(BP, NF)
    M = jnp.kron(jnp.eye(N, dtype=f32), W.T)

    feat, s1, s2 = pl.pallas_call(
        _encode_body,
        grid=(G,),
        in_specs=[pl.BlockSpec((R, NF), lambda i: (i, 0)),
                  pl.BlockSpec((NF, N * C), lambda i: (0, 0))],
        out_specs=[pl.BlockSpec((R, C), lambda i: (i, 0)),
                   pl.BlockSpec((8, NF), lambda i: (0, 0)),
                   pl.BlockSpec((NF, NF), lambda i: (0, 0))],
        out_shape=[jax.ShapeDtypeStruct((BP, C), f32),
                   jax.ShapeDtypeStruct((8, NF), f32),
                   jax.ShapeDtypeStruct((NF, NF), f32)],
    )(x2d, M)

    Mtot = float(B * P * N)
    s1f = s1[0].reshape(N, F).sum(0)
    s2f = jnp.einsum('aiaj->ij', s2.reshape(N, F, N, F))
    mean = (W @ s1f) / Mtot
    ex2 = jnp.einsum('cf,fg,cg->c', W, s2f, W) / Mtot
    var = ex2 - mean * mean
    a = gamma / jnp.sqrt(var + 1e-5)
    bb = beta - mean * a

    col = jnp.clip(pillar_indices[:, :, 2], 0, Xn - 1)
    lin = pillar_indices[:, :, 1] * Xn + col

    sc_dedup_gather = functools.partial(
        pl.kernel,
        mesh=plsc.VectorSubcoreMesh(core_axis_name="c", subcore_axis_name="s"),
        compiler_params=pltpu.CompilerParams(needs_layout_passes=False,
                                             use_tc_tiling_on_sc=False),
        out_type=[jax.ShapeDtypeStruct((B * JW, C), f32),
                  jax.ShapeDtypeStruct((B * JW,), f32)],
        scratch_types=[
            pltpu.VMEM((P,), jnp.int32),
            pltpu.VMEM((16 * JG,), f32),
            pltpu.VMEM((JG,), jnp.int32),
            pltpu.VMEM((JG,), f32),
            pltpu.VMEM((GW, C), f32),
            pltpu.SemaphoreType.DMA,
        ],
    )(_sc_body)
    gathered2d, validw = sc_dedup_gather(lin.reshape(B * P), feat)
    validf = validw.reshape(B * NJW, 1, JT)

    canvas = pl.pallas_call(
        _canvas_body,
        grid=(B, NJT),
        in_specs=[pl.BlockSpec((JT, C), lambda b, j: (b * NJW + jnp.minimum(j, NJW - 1), 0)),
                  pl.BlockSpec((1, 1, JT), lambda b, j: (b * NJW + jnp.minimum(j, NJW - 1), 0, 0)),
                  pl.BlockSpec((C, 1), lambda b, j: (0, 0)),
                  pl.BlockSpec((C, 1), lambda b, j: (0, 0))],
        out_specs=pl.BlockSpec((1, C, JT), lambda b, j: (b, 0, j)),
        out_shape=jax.ShapeDtypeStruct((B, C, J), f32),
    )(gathered2d, validf, a.reshape(C, 1), bb.reshape(C, 1))

    return canvas.reshape(B, C, Xn, Yn)


# speedup vs baseline: 2.8720x; 1.5501x over previous
import functools

import jax
import jax.numpy as jnp
from jax import lax
from jax.experimental import pallas as pl
from jax.experimental.pallas import tpu as pltpu
from jax.experimental.pallas import tpu_sc as plsc

B, P, N, F, C, Xn, Yn = 8, 12000, 32, 9, 64, 144, 496
BP = B * P
NF = N * F
J = Xn * Yn
JW = Xn * Xn
C2 = 128
R = 1200
G = BP // R
PL = P + 32
NGRP = 6
JG = JW // NGRP
JPAD = 3072
JW2 = JW + JPAD
GW = 216
GC = 216
XS = 24
NXB = Xn // XS
NXW = JW2 // (XS * Yn)


def _encode_body(v_ref, m_ref, feat_ref, s1_ref, s2_ref):
    i = pl.program_id(0)
    v = v_ref[...]
    xf = jnp.dot(v, m_ref[...], preferred_element_type=jnp.float32)
    t = xf
    s = 1024
    while s >= 64:
        t = jnp.maximum(t[:, :s], t[:, s:2 * s])
        s //= 2
    feat_ref[...] = jnp.concatenate(
        [t, jnp.zeros((R, C2 - C), jnp.float32)], axis=1)
    ones = jnp.ones((8, R), jnp.bfloat16)
    s1 = jnp.dot(ones, v, preferred_element_type=jnp.float32)
    s2 = jax.lax.dot_general(v, v, (((0,), (0,)), ((), ())),
                             preferred_element_type=jnp.float32)

    @pl.when(i == 0)
    def _():
        s1_ref[...] = jnp.zeros_like(s1_ref)
        s2_ref[...] = jnp.zeros_like(s2_ref)

    s1_ref[...] += s1
    s2_ref[...] += s2


def _sc_body(lin_hbm, feat_hbm, gath_hbm, valid_hbm,
             lin_v, table_v, idx_v, valid_v, gbuf_v, zbuf_v, sem):
    wid = lax.axis_index("s") * 2 + lax.axis_index("c")
    lanes = lax.iota(jnp.int32, 16)

    def _zinit(i, carry):
        zbuf_v[pl.ds(i * 16, 16)] = jnp.zeros((16,), jnp.float32)
        return carry
    lax.fori_loop(0, JPAD // NGRP // 16, _zinit, 0)

    def run_task(t):
        b = t // NGRP
        grp = t - b * NGRP
        lo = grp * JG
        pltpu.sync_copy(lin_hbm.at[pl.ds(b * PL, PL)], lin_v)

        def _init(i, carry):
            table_v[pl.ds(i * 16, 16)] = jnp.full((16,), -1.0, jnp.float32)
            return carry
        lax.fori_loop(0, (16 * JG) // 16, _init, 0)

        def _scat(v, carry):
            linv = lin_v[pl.ds(v * 16, 16)]
            off = linv - lo
            mask = (off >= 0) & (off < JG)
            idx = lanes * JG + jnp.clip(off, 0, JG - 1)
            pval = (v * 16 + lanes).astype(jnp.float32)
            plsc.store_scatter(table_v, [idx], pval, mask=mask)
            return carry
        lax.fori_loop(0, PL // 16, _scat, 0)

        def _merge(i, carry):
            joff = i * 16
            acc = table_v[pl.ds(joff, 16)]
            for l in range(1, 16):
                acc = jnp.maximum(acc, table_v[pl.ds(l * JG + joff, 16)])
            valid_v[pl.ds(joff, 16)] = jnp.where(acc >= 0.0, 1.0, 0.0)
            spread = lax.rem(lo + joff + lanes, P)
            gidx = b * P + jnp.where(acc >= 0.0, acc.astype(jnp.int32), spread)
            idx_v[pl.ds(joff, 16)] = gidx
            return carry
        lax.fori_loop(0, JG // 16, _merge, 0)

        for w in range(JG // GW):
            base = w * GW
            cps = []
            for k in range(GW // GC):
                cp = pltpu.make_async_copy(
                    feat_hbm.at[idx_v.at[pl.ds(base + k * GC, GC)]],
                    gbuf_v.at[pl.ds(k * GC, GC)], sem)
                cp.start()
                cps.append(cp)
            for cp in cps:
                cp.wait()
            pltpu.sync_copy(gbuf_v, gath_hbm.at[pl.ds(b * JW2 + lo + base, GW)])

        pltpu.sync_copy(valid_v, valid_hbm.at[pl.ds(b * JW2 + lo, JG)])
        pltpu.sync_copy(
            zbuf_v, valid_hbm.at[pl.ds(b * JW2 + JW + grp * (JPAD // NGRP),
                                       JPAD // NGRP)])

    run_task(wid)

    @pl.when(wid < B * NGRP - 32)
    def _():
        run_task(wid + 32)


def _canvas_body(g_ref, valid_ref, a_ref, b_ref, out_ref):
    jt = pl.program_id(1)

    @pl.when(jt < NXW)
    def _():
        for x in range(XS):
            g = g_ref[pl.ds(x * Yn, Yn), :]
            gt = g[:, :C].T
            y = jnp.maximum(gt * a_ref[...] + b_ref[...], 0.0)
            m = valid_ref[x]
            out_ref[0, :, x, :] = jnp.where(m > 0.0, y, 0.0)

    @pl.when(jt >= NXW)
    def _():
        out_ref[...] = jnp.zeros_like(out_ref)


def kernel(pillars, pillar_indices, W, gamma, beta):
    f32 = jnp.float32
    x2d = pillars.reshape(BP, NF).astype(jnp.bfloat16)
    M = jnp.kron(jnp.eye(N, dtype=f32), W.T).astype(jnp.bfloat16)

    feat, s1, s2 = pl.pallas_call(
        _encode_body,
        grid=(G,),
        in_specs=[pl.BlockSpec((R, NF), lambda i: (i, 0)),
                  pl.BlockSpec((NF, N * C), lambda i: (0, 0))],
        out_specs=[pl.BlockSpec((R, C2), lambda i: (i, 0)),
                   pl.BlockSpec((8, NF), lambda i: (0, 0)),
                   pl.BlockSpec((NF, NF), lambda i: (0, 0))],
        out_shape=[jax.ShapeDtypeStruct((BP, C2), f32),
                   jax.ShapeDtypeStruct((8, NF), f32),
                   jax.ShapeDtypeStruct((NF, NF), f32)],
    )(x2d, M)

    Mtot = float(B * P * N)
    s1f = s1[0].reshape(N, F).sum(0)
    s2f = jnp.einsum('aiaj->ij', s2.reshape(N, F, N, F))
    mean = (W @ s1f) / Mtot
    ex2 = jnp.einsum('cf,fg,cg->c', W, s2f, W) / Mtot
    var = ex2 - mean * mean
    a = gamma / jnp.sqrt(var + 1e-5)
    bb = beta - mean * a

    col = jnp.clip(pillar_indices[:, :, 2], 0, Xn - 1)
    lin = pillar_indices[:, :, 1] * Xn + col
    lin_pad = jnp.pad(lin, ((0, 0), (0, PL - P)), constant_values=JW)

    sc_dedup_gather = functools.partial(
        pl.kernel,
        mesh=plsc.VectorSubcoreMesh(core_axis_name="c", subcore_axis_name="s"),
        compiler_params=pltpu.CompilerParams(needs_layout_passes=False),
        out_type=[jax.ShapeDtypeStruct((B * JW2, C2), f32),
                  jax.ShapeDtypeStruct((B * JW2,), f32)],
        scratch_types=[
            pltpu.VMEM((PL,), jnp.int32),
            pltpu.VMEM((16 * JG,), f32),
            pltpu.VMEM((JG,), jnp.int32),
            pltpu.VMEM((JG,), f32),
            pltpu.VMEM((GW, C2), f32),
            pltpu.VMEM((JPAD // NGRP,), f32),
            pltpu.SemaphoreType.DMA,
        ],
    )(_sc_body)
    gathered2d, validw = sc_dedup_gather(lin_pad.reshape(B * PL), feat)
    valid3d = validw.reshape(B * (JW2 // Yn), 1, Yn)

    canvas = pl.pallas_call(
        _canvas_body,
        grid=(B, NXB),
        in_specs=[pl.BlockSpec((XS * Yn, C2),
                               lambda b, j: (b * NXW + jnp.minimum(j, NXW - 1), 0)),
                  pl.BlockSpec((XS, 1, Yn),
                               lambda b, j: (b * NXW + jnp.minimum(j, NXW - 1), 0, 0)),
                  pl.BlockSpec((C, 1), lambda b, j: (0, 0)),
                  pl.BlockSpec((C, 1), lambda b, j: (0, 0))],
        out_specs=pl.BlockSpec((1, C, XS, Yn), lambda b, j: (b, 0, j, 0)),
        out_shape=jax.ShapeDtypeStruct((B, C, Xn, Yn), f32),
    )(gathered2d, valid3d, a.reshape(C, 1), bb.reshape(C, 1))

    return canvas
